# Initial kernel scaffold; baseline (speedup 1.0000x reference)
import jax
import jax.numpy as jnp
from jax.experimental import pallas as pl


def kernel(input_ids, embedding):
    raise NotImplementedError("write your pallas kernel here")


# speedup vs baseline: 6.5315x; 6.5315x over previous
import jax
import jax.numpy as jnp
from jax.experimental import pallas as pl
from jax.experimental.pallas import tpu as pltpu
from jax.experimental.pallas import tpu_sc as plsc

WINDOW = 128


def kernel(input_ids, embedding):
    batch, seq = input_ids.shape
    vocab, dim = embedding.shape
    num_idx = batch * seq
    assert num_idx % WINDOW == 0

    idx = input_ids.reshape(1, num_idx).astype(jnp.int32)
    mesh = plsc.VectorSubcoreMesh(core_axis_name="core", subcore_axis_name="subcore")

    @jax.jit
    def gather(table, idx):
        @pl.kernel(
            out_type=jax.ShapeDtypeStruct((num_idx, dim), table.dtype),
            mesh=mesh,
        )
        def gather_kernel(table_hbm, idx_hbm, out_hbm):
            def body(i_vmem, o_vmem):
                pltpu.sync_copy(table_hbm.at[i_vmem.at[0]], o_vmem)

            pltpu.emit_pipeline(
                body,
                grid=(num_idx // WINDOW,),
                in_specs=[pl.BlockSpec((1, WINDOW), lambda i: (0, i))],
                out_specs=[pl.BlockSpec((WINDOW, dim), lambda i: (i, 0))],
                core_axis_name=("core", "subcore"),
                dimension_semantics=(pltpu.PARALLEL,),
            )(idx_hbm, out_hbm)

        return gather_kernel(table, idx)

    out = gather(embedding, idx)
    return out.reshape(batch, seq, dim)


# speedup vs baseline: 7.6911x; 1.1775x over previous
import jax
import jax.numpy as jnp
from jax.experimental import pallas as pl
from jax.experimental.pallas import tpu as pltpu
from jax.experimental.pallas import tpu_sc as plsc

WINDOW = 128
GATHERS_PER_STEP = 2
BLOCK = WINDOW * GATHERS_PER_STEP


def kernel(input_ids, embedding):
    batch, seq = input_ids.shape
    vocab, dim = embedding.shape
    num_idx = batch * seq
    assert num_idx % BLOCK == 0

    idx = input_ids.reshape(1, num_idx).astype(jnp.int32)
    mesh = plsc.VectorSubcoreMesh(core_axis_name="core", subcore_axis_name="subcore")

    @jax.jit
    def gather(table, idx):
        @pl.kernel(
            out_type=jax.ShapeDtypeStruct((num_idx, dim), table.dtype),
            mesh=mesh,
            scratch_types=[pltpu.SemaphoreType.DMA],
        )
        def gather_kernel(table_hbm, idx_hbm, out_hbm, sem):
            def body(i_vmem, o_vmem):
                copies = [
                    pltpu.async_copy(
                        table_hbm.at[i_vmem.at[0, pl.ds(g * WINDOW, WINDOW)]],
                        o_vmem.at[pl.ds(g * WINDOW, WINDOW)],
                        sem,
                    )
                    for g in range(GATHERS_PER_STEP)
                ]
                for c in copies:
                    c.wait()

            pltpu.emit_pipeline(
                body,
                grid=(num_idx // BLOCK,),
                in_specs=[pl.BlockSpec((1, BLOCK), lambda i: (0, i))],
                out_specs=[pl.BlockSpec((BLOCK, dim), lambda i: (i, 0))],
                core_axis_name=("core", "subcore"),
                dimension_semantics=(pltpu.PARALLEL,),
            )(idx_hbm, out_hbm)

        return gather_kernel(table, idx)

    out = gather(embedding, idx)
    return out.reshape(batch, seq, dim)
